# baseline (speedup 1.0000x reference)
import functools

import jax
import jax.numpy as jnp
from jax import lax
from jax.experimental import pallas as pl
from jax.experimental.pallas import tpu as pltpu
from jax.experimental.pallas import tpu_sc as plsc

N = 10000
E = 320000
D = 128
CH = 128
NCHUNK = E // CH
EB = NCHUNK
NW = 32
EPAD = 327680
NKW = EPAD // CH // NW
CH2 = 512
NKW2 = EPAD // CH2 // NW
NPAD = 10240
STRIPE1 = NPAD // 16
ROWS_T = NPAD // 16
R0 = 63999
R1 = 64000
FRAC = 0.80078125
_SIGN = -2147483648
_MASK31 = 2147483647


def _thresh_body(att_ref, out_ref, keys_ref):
    SIGN = jnp.int32(_SIGN)
    MASK31 = jnp.int32(_MASK31)
    a = att_ref[...]
    b = lax.bitcast_convert_type(a, jnp.int32)
    keys = b ^ ((b >> 31) & MASK31)
    keys_ref[...] = keys

    def round_fn(i, uv):
        sh = 31 - i
        bit = lax.shift_left(jnp.int32(1), sh)
        low = bit - jnp.int32(1)
        t = (uv | low) ^ SIGN
        cnt = jnp.sum((keys_ref[...] <= t).astype(jnp.int32))
        return jnp.where(cnt >= jnp.int32(R1 + 1), uv, uv | bit)

    uv = lax.fori_loop(0, 32, round_fn, jnp.int32(0))
    vhi = uv ^ SIGN
    k = keys_ref[...]
    lt = k < vhi
    cnt_lt = jnp.sum(lt.astype(jnp.int32))
    maxbelow = jnp.max(jnp.where(lt, k, SIGN))
    vlo = jnp.where(cnt_lt <= jnp.int32(R0), vhi, maxbelow)

    def untr(z):
        return z ^ ((z >> 31) & MASK31)

    lo_f = lax.bitcast_convert_type(untr(vlo), jnp.float32)
    hi_f = lax.bitcast_convert_type(untr(vhi), jnp.float32)
    thr = lo_f + jnp.float32(FRAC) * (hi_f - lo_f)
    out_ref[...] = jnp.where(a > thr, a, jnp.float32(0.0))


def _worker_id():
    c = lax.axis_index("c")
    s = lax.axis_index("s")
    return c, s, s * 2 + c


def _attsum_body(masked_hbm, src_hbm, out_hbm,
                 mv0, mv1, sp00, sp01, sp02, sp03, sp10, sp11, sp12, sp13,
                 zv, acc, isem0, isem1, ssem0, ssem1):
    c, s, w = _worker_id()
    mv = [mv0, mv1]
    sp = [[sp00, sp01, sp02, sp03], [sp10, sp11, sp12, sp13]]
    isem = [isem0, isem1]
    ssem = [ssem0, ssem1]
    base0 = w * NKW2 * CH2

    for j in range(STRIPE1 // 16):
        zv[pl.ds(16 * j, 16)] = jnp.zeros((16,), jnp.float32)
    pltpu.sync_copy(zv, acc.at[pl.ds(STRIPE1 * s, STRIPE1)])
    plsc.subcore_barrier()

    def issue_loads(k, b):
        base = base0 + k * CH2
        pltpu.async_copy(masked_hbm.at[pl.ds(base, CH2)], mv[b], isem[b])
        for j in range(4):
            pltpu.async_copy(src_hbm.at[pl.ds(base + CH * j, CH)],
                             sp[b][j], isem[b])

    def wait_loads(b):
        pltpu.make_async_copy(masked_hbm.at[pl.ds(0, CH2)], mv[b],
                              isem[b]).wait()
        for j in range(4):
            pltpu.make_async_copy(src_hbm.at[pl.ds(0, CH)], sp[b][j],
                                  isem[b]).wait()

    issue_loads(0, 0)
    issue_loads(1, 1)
    for k in range(NKW2):
        b = k % 2
        wait_loads(b)
        for j in range(4):
            pltpu.async_copy(mv[b].at[pl.ds(CH * j, CH)],
                             acc.at[sp[b][j]], ssem[b], add=True)
        for j in range(4):
            pltpu.make_async_copy(mv[b].at[pl.ds(CH * j, CH)],
                                  acc.at[sp[b][j]], ssem[b]).wait()
        if k + 2 < NKW2:
            issue_loads(k + 2, b)

    plsc.subcore_barrier()
    pltpu.sync_copy(acc.at[pl.ds(STRIPE1 * s, STRIPE1)],
                    out_hbm.at[pl.ds(c * NPAD + STRIPE1 * s, STRIPE1)])


def _attw_body(masked_hbm, src_hbm, s0_hbm, s1_hbm, out_hbm,
               mv0, mv1, sp00, sp01, sp02, sp03, sp10, sp11, sp12, sp13,
               g00, g01, g10, g11, wv0, wv1, isem0, isem1, gsem0, gsem1):
    c, s, w = _worker_id()
    mv = [mv0, mv1]
    sp = [[sp00, sp01, sp02, sp03], [sp10, sp11, sp12, sp13]]
    g0 = [g00, g01]
    g1 = [g10, g11]
    wv = [wv0, wv1]
    isem = [isem0, isem1]
    gsem = [gsem0, gsem1]
    base0 = w * NKW2 * CH2

    def issue_loads(k, b):
        base = base0 + k * CH2
        pltpu.async_copy(masked_hbm.at[pl.ds(base, CH2)], mv[b], isem[b])
        for j in range(4):
            pltpu.async_copy(src_hbm.at[pl.ds(base + CH * j, CH)],
                             sp[b][j], isem[b])

    def wait_loads(b):
        pltpu.make_async_copy(masked_hbm.at[pl.ds(0, CH2)], mv[b],
                              isem[b]).wait()
        for j in range(4):
            pltpu.make_async_copy(src_hbm.at[pl.ds(0, CH)], sp[b][j],
                                  isem[b]).wait()

    issue_loads(0, 0)
    issue_loads(1, 1)
    for k in range(NKW2):
        b = k % 2
        wait_loads(b)
        for j in range(4):
            pltpu.async_copy(s0_hbm.at[sp[b][j]],
                             g0[b].at[pl.ds(CH * j, CH)], gsem[b])
            pltpu.async_copy(s1_hbm.at[sp[b][j]],
                             g1[b].at[pl.ds(CH * j, CH)], gsem[b])
        for j in range(4):
            pltpu.make_async_copy(s0_hbm.at[sp[b][j]],
                                  g0[b].at[pl.ds(CH * j, CH)], gsem[b]).wait()
            pltpu.make_async_copy(s1_hbm.at[sp[b][j]],
                                  g1[b].at[pl.ds(CH * j, CH)], gsem[b]).wait()
        for j in range(CH2 // 16):
            sl = pl.ds(16 * j, 16)
            wv[b][sl] = mv[b][sl] / (g0[b][sl] + g1[b][sl] + jnp.float32(1e-16))
        pltpu.sync_copy(wv[b], out_hbm.at[pl.ds(base0 + k * CH2, CH2)])
        if k + 2 < NKW2:
            issue_loads(k + 2, b)


def _spmv_body(y_hbm, src_hbm, dst_hbm, w_hbm, out_hbm,
               sv0, sv1, dv0, dv1, wv0, wv1, rows0, rows1, acc,
               isem0, isem1, gsem0, gsem1):
    c, s, w = _worker_id()
    sv = [sv0, sv1]
    dv = [dv0, dv1]
    wv = [wv0, wv1]
    rows = [rows0, rows1]
    isem = [isem0, isem1]
    gsem = [gsem0, gsem1]
    base0 = w * NKW * CH

    def zero_body(i, carry):
        for j in range(D // 16):
            rows1[i, pl.ds(16 * j, 16)] = jnp.zeros((16,), jnp.float32)
        return carry

    lax.fori_loop(0, CH, zero_body, jnp.int32(0))
    for j in range(5):
        pltpu.sync_copy(rows1, acc.at[pl.ds(ROWS_T * s + (ROWS_T // 5) * j,
                                            ROWS_T // 5)])
    plsc.subcore_barrier()

    def issue_loads(k, b):
        base = base0 + k * CH
        pltpu.async_copy(src_hbm.at[pl.ds(base, CH)], sv[b], isem[b])
        pltpu.async_copy(dst_hbm.at[pl.ds(base, CH)], dv[b], isem[b])
        pltpu.async_copy(w_hbm.at[pl.ds(base, CH)], wv[b], isem[b])

    def wait_loads(b):
        pltpu.make_async_copy(src_hbm.at[pl.ds(0, CH)], sv[b], isem[b]).wait()
        pltpu.make_async_copy(dst_hbm.at[pl.ds(0, CH)], dv[b], isem[b]).wait()
        pltpu.make_async_copy(w_hbm.at[pl.ds(0, CH)], wv[b], isem[b]).wait()

    def issue_gather(b):
        pltpu.async_copy(y_hbm.at[sv[b]], rows[b], gsem[b])

    def wait_gather(b):
        pltpu.make_async_copy(y_hbm.at[sv[b]], rows[b], gsem[b]).wait()

    def scale(b):
        idx0 = jnp.zeros((16,), jnp.int32)

        def scale_body(i, carry2):
            wb = plsc.load_gather(wv[b], [idx0 + i])
            for j in range(D // 16):
                sl = pl.ds(16 * j, 16)
                rows[b][i, sl] = rows[b][i, sl] * wb
            return carry2

        lax.fori_loop(0, CH, scale_body, jnp.int32(0))

    def iteration(k, b, prefetch_gather, prefetch_loads):
        bn = 1 - b
        wait_gather(b)
        if prefetch_gather:
            wait_loads(bn)
            issue_gather(bn)
        scale(b)
        pltpu.sync_copy(rows[b], acc.at[dv[b]], add=True)
        if prefetch_loads:
            issue_loads(k + 2, b)

    issue_loads(0, 0)
    wait_loads(0)
    issue_gather(0)
    issue_loads(1, 1)

    def pair_body(pk, carry):
        k0 = 2 * pk
        iteration(k0, 0, True, True)
        iteration(k0 + 1, 1, True, True)
        return carry

    lax.fori_loop(0, (NKW - 2) // 2, pair_body, jnp.int32(0))
    iteration(NKW - 2, 0, True, False)
    iteration(NKW - 1, 1, False, False)

    plsc.subcore_barrier()
    pltpu.sync_copy(acc.at[pl.ds(ROWS_T * s, ROWS_T)],
                    out_hbm.at[pl.ds(c * NPAD + ROWS_T * s, ROWS_T)])


def _combine_body(p0_ref, p1_ref, p_ref, z_ref, pn_ref, zn_ref, *, coef):
    pn = p0_ref[...] + p1_ref[...] - p_ref[...]
    pn_ref[...] = pn
    zn_ref[...] = z_ref[...] + jnp.float32(coef) * pn


_CACHE = {}


def _cached(name, builder):
    if name not in _CACHE:
        _CACHE[name] = builder()
    return _CACHE[name]


def _build_thresh():
    return pl.pallas_call(
        _thresh_body,
        out_shape=jax.ShapeDtypeStruct((EB, CH), jnp.float32),
        scratch_shapes=[pltpu.VMEM((EB, CH), jnp.int32)],
    )


def _mesh():
    return plsc.VectorSubcoreMesh(core_axis_name="c", subcore_axis_name="s")


def _build_attsum():
    return pl.kernel(
        _attsum_body,
        out_type=jax.ShapeDtypeStruct((2 * NPAD,), jnp.float32),
        mesh=_mesh(),
        scratch_types=(
            [pltpu.VMEM((CH2,), jnp.float32)] * 2
            + [pltpu.VMEM((CH,), jnp.int32)] * 8
            + [pltpu.VMEM((STRIPE1,), jnp.float32),
               pltpu.VMEM_SHARED((NPAD,), jnp.float32)]
            + [pltpu.SemaphoreType.DMA] * 4
        ),
    )


def _build_attw():
    return pl.kernel(
        _attw_body,
        out_type=jax.ShapeDtypeStruct((EPAD,), jnp.float32),
        mesh=_mesh(),
        scratch_types=(
            [pltpu.VMEM((CH2,), jnp.float32)] * 2
            + [pltpu.VMEM((CH,), jnp.int32)] * 8
            + [pltpu.VMEM((CH2,), jnp.float32)] * 6
            + [pltpu.SemaphoreType.DMA] * 4
        ),
    )


def _build_spmv():
    return pl.kernel(
        _spmv_body,
        out_type=jax.ShapeDtypeStruct((2 * NPAD, D), jnp.float32),
        mesh=_mesh(),
        scratch_types=(
            [pltpu.VMEM((CH,), jnp.int32)] * 4
            + [pltpu.VMEM((CH,), jnp.float32)] * 2
            + [pltpu.VMEM((CH, D), jnp.float32)] * 2
            + [pltpu.VMEM_SHARED((NPAD, D), jnp.float32)]
            + [pltpu.SemaphoreType.DMA] * 4
        ),
        compiler_params=pltpu.CompilerParams(needs_layout_passes=False),
    )


def _build_combine(coef):
    blk = 80
    bs = pl.BlockSpec((blk, D), lambda i: (i, 0))
    bs_p1 = pl.BlockSpec((blk, D), lambda i: (i + NPAD // blk, 0))
    return pl.pallas_call(
        functools.partial(_combine_body, coef=coef),
        grid=(N // blk,),
        in_specs=[bs, bs_p1, bs, bs],
        out_specs=[bs, bs],
        out_shape=[jax.ShapeDtypeStruct((N, D), jnp.float32)] * 2,
    )


def kernel(x, edge_index, att):
    src = jnp.concatenate([edge_index[0], jnp.zeros((EPAD - E,), jnp.int32)])
    dst = jnp.concatenate([edge_index[1], jnp.zeros((EPAD - E,), jnp.int32)])
    att2d = att.reshape(EB, CH)
    masked = jnp.concatenate(
        [_cached("thresh", _build_thresh)(att2d).reshape(E),
         jnp.zeros((EPAD - E,), jnp.float32)])
    sums = _cached("attsum", _build_attsum)(masked, src)
    attw = _cached("attw", _build_attw)(masked, src,
                                        sums[:NPAD], sums[NPAD:])
    p = x
    z = x
    for i, coef in enumerate((1.0, 0.5, 1.0 / 6.0, 1.0 / 24.0)):
        parts = _cached("spmv", _build_spmv)(p, src, dst, attw)
        p, z = _cached(f"combine{i}", lambda: _build_combine(coef))(
            parts, parts, p, z)
    return z


# speedup vs baseline: 1.8982x; 1.8982x over previous
import functools

import jax
import jax.numpy as jnp
from jax import lax
from jax.experimental import pallas as pl
from jax.experimental.pallas import tpu as pltpu
from jax.experimental.pallas import tpu_sc as plsc

N = 10000
E = 320000
D = 128
CH = 128
NCHUNK = E // CH
EB = NCHUNK
NW = 32
EPAD = 327680
NKW = EPAD // CH // NW
CH2 = 512
NKW2 = EPAD // CH2 // NW
NPAD = 10240
STRIPE1 = NPAD // 16
ROWS_T = NPAD // 16
R0 = 63999
R1 = 64000
FRAC = 0.80078125
_SIGN = -2147483648
_MASK31 = 2147483647


def _thresh_body(att_ref, out_ref, keys_ref):
    SIGN = jnp.int32(_SIGN)
    MASK31 = jnp.int32(_MASK31)
    a = att_ref[...]
    b = lax.bitcast_convert_type(a, jnp.int32)
    keys = b ^ ((b >> 31) & MASK31)
    keys_ref[...] = keys

    def round_fn(i, uv):
        sh = 31 - i
        bit = lax.shift_left(jnp.int32(1), sh)
        low = bit - jnp.int32(1)
        t = (uv | low) ^ SIGN
        cnt = jnp.sum((keys_ref[...] <= t).astype(jnp.int32))
        return jnp.where(cnt >= jnp.int32(R1 + 1), uv, uv | bit)

    uv = lax.fori_loop(0, 32, round_fn, jnp.int32(0))
    vhi = uv ^ SIGN
    k = keys_ref[...]
    lt = k < vhi
    cnt_lt = jnp.sum(lt.astype(jnp.int32))
    maxbelow = jnp.max(jnp.where(lt, k, SIGN))
    vlo = jnp.where(cnt_lt <= jnp.int32(R0), vhi, maxbelow)

    def untr(z):
        return z ^ ((z >> 31) & MASK31)

    lo_f = lax.bitcast_convert_type(untr(vlo), jnp.float32)
    hi_f = lax.bitcast_convert_type(untr(vhi), jnp.float32)
    thr = lo_f + jnp.float32(FRAC) * (hi_f - lo_f)
    out_ref[...] = jnp.where(a > thr, a, jnp.float32(0.0))


def _worker_id():
    c = lax.axis_index("c")
    s = lax.axis_index("s")
    return c, s, s * 2 + c


def _attsum_body(masked_hbm, src_hbm, out_hbm,
                 mv0, mv1, sp00, sp01, sp02, sp03, sp10, sp11, sp12, sp13,
                 zv, acc, isem0, isem1, ssem0, ssem1):
    c, s, w = _worker_id()
    mv = [mv0, mv1]
    sp = [[sp00, sp01, sp02, sp03], [sp10, sp11, sp12, sp13]]
    isem = [isem0, isem1]
    ssem = [ssem0, ssem1]
    base0 = w * NKW2 * CH2

    for j in range(STRIPE1 // 16):
        zv[pl.ds(16 * j, 16)] = jnp.zeros((16,), jnp.float32)
    pltpu.sync_copy(zv, acc.at[pl.ds(STRIPE1 * s, STRIPE1)])
    plsc.subcore_barrier()

    def issue_loads(k, b):
        base = base0 + k * CH2
        pltpu.async_copy(masked_hbm.at[pl.ds(base, CH2)], mv[b], isem[b])
        for j in range(4):
            pltpu.async_copy(src_hbm.at[pl.ds(base + CH * j, CH)],
                             sp[b][j], isem[b])

    def wait_loads(b):
        pltpu.make_async_copy(masked_hbm.at[pl.ds(0, CH2)], mv[b],
                              isem[b]).wait()
        for j in range(4):
            pltpu.make_async_copy(src_hbm.at[pl.ds(0, CH)], sp[b][j],
                                  isem[b]).wait()

    issue_loads(0, 0)
    issue_loads(1, 1)
    for k in range(NKW2):
        b = k % 2
        wait_loads(b)
        for j in range(4):
            pltpu.async_copy(mv[b].at[pl.ds(CH * j, CH)],
                             acc.at[sp[b][j]], ssem[b], add=True)
        for j in range(4):
            pltpu.make_async_copy(mv[b].at[pl.ds(CH * j, CH)],
                                  acc.at[sp[b][j]], ssem[b]).wait()
        if k + 2 < NKW2:
            issue_loads(k + 2, b)

    plsc.subcore_barrier()
    pltpu.sync_copy(acc.at[pl.ds(STRIPE1 * s, STRIPE1)],
                    out_hbm.at[pl.ds(c * NPAD + STRIPE1 * s, STRIPE1)])


def _attw_body(masked_hbm, src_hbm, s0_hbm, s1_hbm, out_hbm,
               mv0, mv1, sp00, sp01, sp02, sp03, sp10, sp11, sp12, sp13,
               g00, g01, g10, g11, wv0, wv1, isem0, isem1, gsem0, gsem1):
    c, s, w = _worker_id()
    mv = [mv0, mv1]
    sp = [[sp00, sp01, sp02, sp03], [sp10, sp11, sp12, sp13]]
    g0 = [g00, g01]
    g1 = [g10, g11]
    wv = [wv0, wv1]
    isem = [isem0, isem1]
    gsem = [gsem0, gsem1]
    base0 = w * NKW2 * CH2

    def issue_loads(k, b):
        base = base0 + k * CH2
        pltpu.async_copy(masked_hbm.at[pl.ds(base, CH2)], mv[b], isem[b])
        for j in range(4):
            pltpu.async_copy(src_hbm.at[pl.ds(base + CH * j, CH)],
                             sp[b][j], isem[b])

    def wait_loads(b):
        pltpu.make_async_copy(masked_hbm.at[pl.ds(0, CH2)], mv[b],
                              isem[b]).wait()
        for j in range(4):
            pltpu.make_async_copy(src_hbm.at[pl.ds(0, CH)], sp[b][j],
                                  isem[b]).wait()

    issue_loads(0, 0)
    issue_loads(1, 1)
    for k in range(NKW2):
        b = k % 2
        wait_loads(b)
        for j in range(4):
            pltpu.async_copy(s0_hbm.at[sp[b][j]],
                             g0[b].at[pl.ds(CH * j, CH)], gsem[b])
            pltpu.async_copy(s1_hbm.at[sp[b][j]],
                             g1[b].at[pl.ds(CH * j, CH)], gsem[b])
        for j in range(4):
            pltpu.make_async_copy(s0_hbm.at[sp[b][j]],
                                  g0[b].at[pl.ds(CH * j, CH)], gsem[b]).wait()
            pltpu.make_async_copy(s1_hbm.at[sp[b][j]],
                                  g1[b].at[pl.ds(CH * j, CH)], gsem[b]).wait()
        for j in range(CH2 // 16):
            sl = pl.ds(16 * j, 16)
            wv[b][sl] = mv[b][sl] / (g0[b][sl] + g1[b][sl] + jnp.float32(1e-16))
        pltpu.sync_copy(wv[b], out_hbm.at[pl.ds(base0 + k * CH2, CH2)])
        if k + 2 < NKW2:
            issue_loads(k + 2, b)


def _spmv_body(y_hbm, src_hbm, dst_hbm, w_hbm, out_hbm,
               sv0, sv1, dv0, dv1, wv0, wv1, rows0, rows1, acc,
               isem0, isem1, gsem0, gsem1):
    c, s, w = _worker_id()
    sv = [sv0, sv1]
    dv = [dv0, dv1]
    wv = [wv0, wv1]
    rows = [rows0, rows1]
    isem = [isem0, isem1]
    gsem = [gsem0, gsem1]
    base0 = w * NKW * CH

    def zero_body(i, carry):
        for j in range(D // 16):
            rows1[i, pl.ds(16 * j, 16)] = jnp.zeros((16,), jnp.float32)
        return carry

    lax.fori_loop(0, CH, zero_body, jnp.int32(0))
    for j in range(5):
        pltpu.sync_copy(rows1, acc.at[pl.ds(ROWS_T * s + (ROWS_T // 5) * j,
                                            ROWS_T // 5)])
    plsc.subcore_barrier()

    def issue_loads(k, b):
        base = base0 + k * CH
        pltpu.async_copy(src_hbm.at[pl.ds(base, CH)], sv[b], isem[b])
        pltpu.async_copy(dst_hbm.at[pl.ds(base, CH)], dv[b], isem[b])
        pltpu.async_copy(w_hbm.at[pl.ds(base, CH)], wv[b], isem[b])

    def wait_loads(b):
        pltpu.make_async_copy(src_hbm.at[pl.ds(0, CH)], sv[b], isem[b]).wait()
        pltpu.make_async_copy(dst_hbm.at[pl.ds(0, CH)], dv[b], isem[b]).wait()
        pltpu.make_async_copy(w_hbm.at[pl.ds(0, CH)], wv[b], isem[b]).wait()

    def issue_gather(b):
        pltpu.async_copy(y_hbm.at[sv[b]], rows[b], gsem[b])

    def wait_gather(b):
        pltpu.make_async_copy(y_hbm.at[sv[b]], rows[b], gsem[b]).wait()

    def scale(b):
        idx0 = jnp.zeros((16,), jnp.int32)

        def scale_body(i, carry2):
            wb = plsc.load_gather(wv[b], [idx0 + i])
            for j in range(D // 16):
                sl = pl.ds(16 * j, 16)
                rows[b][i, sl] = rows[b][i, sl] * wb
            return carry2

        lax.fori_loop(0, CH, scale_body, jnp.int32(0))

    def iteration(k, b, prefetch_gather, prefetch_loads):
        bn = 1 - b
        wait_gather(b)
        if prefetch_gather:
            wait_loads(bn)
            issue_gather(bn)
        scale(b)
        pltpu.sync_copy(rows[b], acc.at[dv[b]], add=True)
        if prefetch_loads:
            issue_loads(k + 2, b)

    issue_loads(0, 0)
    wait_loads(0)
    issue_gather(0)
    issue_loads(1, 1)

    def pair_body(pk, carry):
        k0 = 2 * pk
        iteration(k0, 0, True, True)
        iteration(k0 + 1, 1, True, True)
        return carry

    lax.fori_loop(0, (NKW - 2) // 2, pair_body, jnp.int32(0))
    iteration(NKW - 2, 0, True, False)
    iteration(NKW - 1, 1, False, False)

    plsc.subcore_barrier()
    pltpu.sync_copy(acc.at[pl.ds(ROWS_T * s, ROWS_T)],
                    out_hbm.at[pl.ds(c * NPAD + ROWS_T * s, ROWS_T)])


def _combine_body(p0_ref, p1_ref, p_ref, z_ref, pn_ref, zn_ref, *, coef):
    pn = p0_ref[...] + p1_ref[...] - p_ref[...]
    pn_ref[...] = pn
    zn_ref[...] = z_ref[...] + jnp.float32(coef) * pn


_CACHE = {}


def _cached(name, builder):
    if name not in _CACHE:
        _CACHE[name] = builder()
    return _CACHE[name]


def _build_thresh():
    return pl.pallas_call(
        _thresh_body,
        out_shape=jax.ShapeDtypeStruct((EB, CH), jnp.float32),
        scratch_shapes=[pltpu.VMEM((EB, CH), jnp.int32)],
    )


def _mesh():
    return plsc.VectorSubcoreMesh(core_axis_name="c", subcore_axis_name="s")


def _build_attsum():
    return pl.kernel(
        _attsum_body,
        out_type=jax.ShapeDtypeStruct((2 * NPAD,), jnp.float32),
        mesh=_mesh(),
        scratch_types=(
            [pltpu.VMEM((CH2,), jnp.float32)] * 2
            + [pltpu.VMEM((CH,), jnp.int32)] * 8
            + [pltpu.VMEM((STRIPE1,), jnp.float32),
               pltpu.VMEM_SHARED((NPAD,), jnp.float32)]
            + [pltpu.SemaphoreType.DMA] * 4
        ),
    )


def _build_attw():
    return pl.kernel(
        _attw_body,
        out_type=jax.ShapeDtypeStruct((EPAD,), jnp.float32),
        mesh=_mesh(),
        scratch_types=(
            [pltpu.VMEM((CH2,), jnp.float32)] * 2
            + [pltpu.VMEM((CH,), jnp.int32)] * 8
            + [pltpu.VMEM((CH2,), jnp.float32)] * 6
            + [pltpu.SemaphoreType.DMA] * 4
        ),
    )


def _build_spmv():
    return pl.kernel(
        _spmv_body,
        out_type=jax.ShapeDtypeStruct((2 * NPAD, D), jnp.float32),
        mesh=_mesh(),
        scratch_types=(
            [pltpu.VMEM((CH,), jnp.int32)] * 4
            + [pltpu.VMEM((CH,), jnp.float32)] * 2
            + [pltpu.VMEM((CH, D), jnp.float32)] * 2
            + [pltpu.VMEM_SHARED((NPAD, D), jnp.float32)]
            + [pltpu.SemaphoreType.DMA] * 4
        ),
        compiler_params=pltpu.CompilerParams(needs_layout_passes=False),
    )


def _build_combine(coef):
    blk = 80
    bs = pl.BlockSpec((blk, D), lambda i: (i, 0))
    bs_p1 = pl.BlockSpec((blk, D), lambda i: (i + NPAD // blk, 0))
    return pl.pallas_call(
        functools.partial(_combine_body, coef=coef),
        grid=(N // blk,),
        in_specs=[bs, bs_p1, bs, bs],
        out_specs=[bs, bs],
        out_shape=[jax.ShapeDtypeStruct((N, D), jnp.float32)] * 2,
    )


def kernel(x, edge_index, att):
    pad_idx = jnp.arange(EPAD - E, dtype=jnp.int32)
    src = jnp.concatenate([edge_index[0], pad_idx])
    dst = jnp.concatenate([edge_index[1], pad_idx])
    att2d = att.reshape(EB, CH)
    masked = jnp.concatenate(
        [_cached("thresh", _build_thresh)(att2d).reshape(E),
         jnp.zeros((EPAD - E,), jnp.float32)])
    sums = _cached("attsum", _build_attsum)(masked, src)
    attw = _cached("attw", _build_attw)(masked, src,
                                        sums[:NPAD], sums[NPAD:])
    p = x
    z = x
    for i, coef in enumerate((1.0, 0.5, 1.0 / 6.0, 1.0 / 24.0)):
        parts = _cached("spmv", _build_spmv)(p, src, dst, attw)
        p, z = _cached(f"combine{i}", lambda: _build_combine(coef))(
            parts, parts, p, z)
    return z


# speedup vs baseline: 2.2609x; 1.1911x over previous
import functools

import jax
import jax.numpy as jnp
from jax import lax
from jax.experimental import pallas as pl
from jax.experimental.pallas import tpu as pltpu
from jax.experimental.pallas import tpu_sc as plsc

N = 10000
E = 320000
D = 128
CH = 128
NCHUNK = E // CH
EB = NCHUNK
NW = 32
EPAD = 327680
CHS = 112
NKWS = 90
CH2 = 512
NKW2 = EPAD // CH2 // NW
NPAD = 10240
STRIPE1 = NPAD // 16
ROWS_T = NPAD // 16
R0 = 63999
R1 = 64000
FRAC = 0.80078125
_SIGN = -2147483648
_MASK31 = 2147483647


def _thresh_body(att_ref, out_ref, keys_ref):
    SIGN = jnp.int32(_SIGN)
    MASK31 = jnp.int32(_MASK31)
    a = att_ref[...]
    b = lax.bitcast_convert_type(a, jnp.int32)
    keys = b ^ ((b >> 31) & MASK31)
    keys_ref[...] = keys

    def round_fn(i, uv):
        sh = 31 - i
        bit = lax.shift_left(jnp.int32(1), sh)
        low = bit - jnp.int32(1)
        t = (uv | low) ^ SIGN
        cnt = jnp.sum((keys_ref[...] <= t).astype(jnp.int32))
        return jnp.where(cnt >= jnp.int32(R1 + 1), uv, uv | bit)

    uv = lax.fori_loop(0, 32, round_fn, jnp.int32(0))
    vhi = uv ^ SIGN
    k = keys_ref[...]
    lt = k < vhi
    cnt_lt = jnp.sum(lt.astype(jnp.int32))
    maxbelow = jnp.max(jnp.where(lt, k, SIGN))
    vlo = jnp.where(cnt_lt <= jnp.int32(R0), vhi, maxbelow)

    def untr(z):
        return z ^ ((z >> 31) & MASK31)

    lo_f = lax.bitcast_convert_type(untr(vlo), jnp.float32)
    hi_f = lax.bitcast_convert_type(untr(vhi), jnp.float32)
    thr = lo_f + jnp.float32(FRAC) * (hi_f - lo_f)
    out_ref[...] = jnp.where(a > thr, a, jnp.float32(0.0))


def _worker_id():
    c = lax.axis_index("c")
    s = lax.axis_index("s")
    return c, s, s * 2 + c


def _attsum_body(masked_hbm, src_hbm, out_hbm,
                 mv0, mv1, sp00, sp01, sp02, sp03, sp10, sp11, sp12, sp13,
                 zv, acc, isem0, isem1, ssem0, ssem1):
    c, s, w = _worker_id()
    mv = [mv0, mv1]
    sp = [[sp00, sp01, sp02, sp03], [sp10, sp11, sp12, sp13]]
    isem = [isem0, isem1]
    ssem = [ssem0, ssem1]
    base0 = w * NKW2 * CH2

    for j in range(STRIPE1 // 16):
        zv[pl.ds(16 * j, 16)] = jnp.zeros((16,), jnp.float32)
    pltpu.sync_copy(zv, acc.at[pl.ds(STRIPE1 * s, STRIPE1)])
    plsc.subcore_barrier()

    def issue_loads(k, b):
        base = base0 + k * CH2
        pltpu.async_copy(masked_hbm.at[pl.ds(base, CH2)], mv[b], isem[b])
        for j in range(4):
            pltpu.async_copy(src_hbm.at[pl.ds(base + CH * j, CH)],
                             sp[b][j], isem[b])

    def wait_loads(b):
        pltpu.make_async_copy(masked_hbm.at[pl.ds(0, CH2)], mv[b],
                              isem[b]).wait()
        for j in range(4):
            pltpu.make_async_copy(src_hbm.at[pl.ds(0, CH)], sp[b][j],
                                  isem[b]).wait()

    issue_loads(0, 0)
    issue_loads(1, 1)
    for k in range(NKW2):
        b = k % 2
        wait_loads(b)
        for j in range(4):
            pltpu.async_copy(mv[b].at[pl.ds(CH * j, CH)],
                             acc.at[sp[b][j]], ssem[b], add=True)
        for j in range(4):
            pltpu.make_async_copy(mv[b].at[pl.ds(CH * j, CH)],
                                  acc.at[sp[b][j]], ssem[b]).wait()
        if k + 2 < NKW2:
            issue_loads(k + 2, b)

    plsc.subcore_barrier()
    pltpu.sync_copy(acc.at[pl.ds(STRIPE1 * s, STRIPE1)],
                    out_hbm.at[pl.ds(c * NPAD + STRIPE1 * s, STRIPE1)])


def _attw_body(masked_hbm, src_hbm, s0_hbm, s1_hbm, out_hbm,
               mv0, mv1, sp00, sp01, sp02, sp03, sp10, sp11, sp12, sp13,
               g00, g01, g10, g11, wv0, wv1, isem0, isem1, gsem0, gsem1):
    c, s, w = _worker_id()
    mv = [mv0, mv1]
    sp = [[sp00, sp01, sp02, sp03], [sp10, sp11, sp12, sp13]]
    g0 = [g00, g01]
    g1 = [g10, g11]
    wv = [wv0, wv1]
    isem = [isem0, isem1]
    gsem = [gsem0, gsem1]
    base0 = w * NKW2 * CH2

    def issue_loads(k, b):
        base = base0 + k * CH2
        pltpu.async_copy(masked_hbm.at[pl.ds(base, CH2)], mv[b], isem[b])
        for j in range(4):
            pltpu.async_copy(src_hbm.at[pl.ds(base + CH * j, CH)],
                             sp[b][j], isem[b])

    def wait_loads(b):
        pltpu.make_async_copy(masked_hbm.at[pl.ds(0, CH2)], mv[b],
                              isem[b]).wait()
        for j in range(4):
            pltpu.make_async_copy(src_hbm.at[pl.ds(0, CH)], sp[b][j],
                                  isem[b]).wait()

    issue_loads(0, 0)
    issue_loads(1, 1)
    for k in range(NKW2):
        b = k % 2
        wait_loads(b)
        for j in range(4):
            pltpu.async_copy(s0_hbm.at[sp[b][j]],
                             g0[b].at[pl.ds(CH * j, CH)], gsem[b])
            pltpu.async_copy(s1_hbm.at[sp[b][j]],
                             g1[b].at[pl.ds(CH * j, CH)], gsem[b])
        for j in range(4):
            pltpu.make_async_copy(s0_hbm.at[sp[b][j]],
                                  g0[b].at[pl.ds(CH * j, CH)], gsem[b]).wait()
            pltpu.make_async_copy(s1_hbm.at[sp[b][j]],
                                  g1[b].at[pl.ds(CH * j, CH)], gsem[b]).wait()
        for j in range(CH2 // 16):
            sl = pl.ds(16 * j, 16)
            wv[b][sl] = mv[b][sl] / (g0[b][sl] + g1[b][sl] + jnp.float32(1e-16))
        pltpu.sync_copy(wv[b], out_hbm.at[pl.ds(base0 + k * CH2, CH2)])
        if k + 2 < NKW2:
            issue_loads(k + 2, b)


def _spmv_body(y_hbm, src_hbm, dst_hbm, w_hbm, out_hbm,
               sv0, sv1, sv2, dv0, dv1, dv2, wv0, wv1, wv2,
               rows0, rows1, rows2, acc,
               isem0, isem1, isem2, gsem0, gsem1, gsem2,
               ssem0, ssem1, ssem2):
    c, s, w = _worker_id()
    sv = [sv0, sv1, sv2]
    dv = [dv0, dv1, dv2]
    wv = [wv0, wv1, wv2]
    rows = [rows0, rows1, rows2]
    isem = [isem0, isem1, isem2]
    gsem = [gsem0, gsem1, gsem2]
    ssem = [ssem0, ssem1, ssem2]
    base0 = w * NKWS * CHS

    def zero_body(i, carry):
        for j in range(D // 16):
            rows0[i, pl.ds(16 * j, 16)] = jnp.zeros((16,), jnp.float32)
        return carry

    lax.fori_loop(0, CHS, zero_body, jnp.int32(0))
    for j in range(5):
        pltpu.sync_copy(rows0, acc.at[pl.ds(ROWS_T * s + CHS * j, CHS)])
    pltpu.sync_copy(rows0.at[pl.ds(0, ROWS_T - 5 * CHS)],
                    acc.at[pl.ds(ROWS_T * s + 5 * CHS, ROWS_T - 5 * CHS)])
    plsc.subcore_barrier()

    def issue_loads(k, u):
        base = base0 + k * CHS
        pltpu.async_copy(src_hbm.at[pl.ds(base, CHS)], sv[u], isem[u])
        pltpu.async_copy(dst_hbm.at[pl.ds(base, CHS)], dv[u], isem[u])
        pltpu.async_copy(w_hbm.at[pl.ds(base, CHS)], wv[u], isem[u])

    def wait_loads(u):
        pltpu.make_async_copy(src_hbm.at[pl.ds(0, CHS)], sv[u], isem[u]).wait()
        pltpu.make_async_copy(dst_hbm.at[pl.ds(0, CHS)], dv[u], isem[u]).wait()
        pltpu.make_async_copy(w_hbm.at[pl.ds(0, CHS)], wv[u], isem[u]).wait()

    def issue_gather(u):
        pltpu.async_copy(y_hbm.at[sv[u]], rows[u], gsem[u])

    def wait_gather(u):
        pltpu.make_async_copy(y_hbm.at[sv[u]], rows[u], gsem[u]).wait()

    def issue_scatter(u):
        pltpu.async_copy(rows[u], acc.at[dv[u]], ssem[u], add=True)

    def wait_scatter(u):
        pltpu.make_async_copy(rows[u], acc.at[dv[u]], ssem[u]).wait()

    def scale(u):
        idx0 = jnp.zeros((16,), jnp.int32)

        def scale_body(i, carry2):
            for t in range(4):
                e = i * 4 + t
                wb = plsc.load_gather(wv[u], [idx0 + e])
                for j in range(D // 16):
                    sl = pl.ds(16 * j, 16)
                    rows[u][e, sl] = rows[u][e, sl] * wb
            return carry2

        lax.fori_loop(0, CHS // 4, scale_body, jnp.int32(0))

    def iteration(k, u, wait_sc, pf_gather, pf_loads):
        un = (u + 1) % 3
        up = (u + 2) % 3
        wait_gather(u)
        if wait_sc:
            wait_scatter(up)
        if pf_gather:
            wait_loads(un)
            issue_gather(un)
        scale(u)
        issue_scatter(u)
        if pf_loads:
            issue_loads(k + 2, up)

    issue_loads(0, 0)
    issue_loads(1, 1)
    wait_loads(0)
    issue_gather(0)

    iteration(0, 0, False, True, True)

    def triple_body(t, carry):
        k = 3 * t + 1
        iteration(k, 1, True, True, True)
        iteration(k + 1, 2, True, True, True)
        iteration(k + 2, 0, True, True, True)
        return carry

    lax.fori_loop(0, (NKWS - 3) // 3, triple_body, jnp.int32(0))
    iteration(NKWS - 2, (NKWS - 2) % 3, True, True, False)
    iteration(NKWS - 1, (NKWS - 1) % 3, True, False, False)
    wait_scatter((NKWS - 1) % 3)

    plsc.subcore_barrier()
    pltpu.sync_copy(acc.at[pl.ds(ROWS_T * s, ROWS_T)],
                    out_hbm.at[pl.ds(c * NPAD + ROWS_T * s, ROWS_T)])


def _combine_body(p0_ref, p1_ref, p_ref, z_ref, pn_ref, zn_ref, *, coef):
    pn = p0_ref[...] + p1_ref[...] - p_ref[...]
    pn_ref[...] = pn
    zn_ref[...] = z_ref[...] + jnp.float32(coef) * pn


_CACHE = {}


def _cached(name, builder):
    if name not in _CACHE:
        _CACHE[name] = builder()
    return _CACHE[name]


def _build_thresh():
    return pl.pallas_call(
        _thresh_body,
        out_shape=jax.ShapeDtypeStruct((EB, CH), jnp.float32),
        scratch_shapes=[pltpu.VMEM((EB, CH), jnp.int32)],
    )


def _mesh():
    return plsc.VectorSubcoreMesh(core_axis_name="c", subcore_axis_name="s")


def _build_attsum():
    return pl.kernel(
        _attsum_body,
        out_type=jax.ShapeDtypeStruct((2 * NPAD,), jnp.float32),
        mesh=_mesh(),
        scratch_types=(
            [pltpu.VMEM((CH2,), jnp.float32)] * 2
            + [pltpu.VMEM((CH,), jnp.int32)] * 8
            + [pltpu.VMEM((STRIPE1,), jnp.float32),
               pltpu.VMEM_SHARED((NPAD,), jnp.float32)]
            + [pltpu.SemaphoreType.DMA] * 4
        ),
    )


def _build_attw():
    return pl.kernel(
        _attw_body,
        out_type=jax.ShapeDtypeStruct((EPAD,), jnp.float32),
        mesh=_mesh(),
        scratch_types=(
            [pltpu.VMEM((CH2,), jnp.float32)] * 2
            + [pltpu.VMEM((CH,), jnp.int32)] * 8
            + [pltpu.VMEM((CH2,), jnp.float32)] * 6
            + [pltpu.SemaphoreType.DMA] * 4
        ),
    )


def _build_spmv():
    return pl.kernel(
        _spmv_body,
        out_type=jax.ShapeDtypeStruct((2 * NPAD, D), jnp.float32),
        mesh=_mesh(),
        scratch_types=(
            [pltpu.VMEM((CHS,), jnp.int32)] * 6
            + [pltpu.VMEM((CHS,), jnp.float32)] * 3
            + [pltpu.VMEM((CHS, D), jnp.float32)] * 3
            + [pltpu.VMEM_SHARED((NPAD, D), jnp.float32)]
            + [pltpu.SemaphoreType.DMA] * 9
        ),
        compiler_params=pltpu.CompilerParams(needs_layout_passes=False),
    )


def _build_combine(coef):
    blk = 80
    bs = pl.BlockSpec((blk, D), lambda i: (i, 0))
    bs_p1 = pl.BlockSpec((blk, D), lambda i: (i + NPAD // blk, 0))
    return pl.pallas_call(
        functools.partial(_combine_body, coef=coef),
        grid=(N // blk,),
        in_specs=[bs, bs_p1, bs, bs],
        out_specs=[bs, bs],
        out_shape=[jax.ShapeDtypeStruct((N, D), jnp.float32)] * 2,
    )


def kernel(x, edge_index, att):
    pad_idx = jnp.arange(EPAD - E, dtype=jnp.int32)
    src = jnp.concatenate([edge_index[0], pad_idx])
    dst = jnp.concatenate([edge_index[1], pad_idx])
    att2d = att.reshape(EB, CH)
    masked = jnp.concatenate(
        [_cached("thresh", _build_thresh)(att2d).reshape(E),
         jnp.zeros((EPAD - E,), jnp.float32)])
    sums = _cached("attsum", _build_attsum)(masked, src)
    attw = _cached("attw", _build_attw)(masked, src,
                                        sums[:NPAD], sums[NPAD:])
    p = x
    z = x
    for i, coef in enumerate((1.0, 0.5, 1.0 / 6.0, 1.0 / 24.0)):
        parts = _cached("spmv", _build_spmv)(p, src, dst, attw)
        p, z = _cached(f"combine{i}", lambda: _build_combine(coef))(
            parts, parts, p, z)
    return z
